# baseline (speedup 1.0000x reference)
import functools

import jax
import jax.numpy as jnp
from jax import lax
from jax.experimental import pallas as pl
from jax.experimental.pallas import tpu as pltpu
from jax.experimental.pallas import tpu_sc as plsc

_VOCAB = 1000000
_VPAD = 1048576
_HID = 64
_OUT = 2
_B = 4096
_L = 200

_NC = 2
_NS = 16
_NW = _NC * _NS
_IPW = _B // _NW
_C0 = 104
_C1 = _L - _C0
_UNROLL = 8

_PROJ_BLK = 32768


def _tc_project(table_t, fc1_w):
    def body(w_ref, x_ref, o_ref):
        o_ref[...] = jax.lax.dot_general(
            w_ref[...], x_ref[...], (((1,), (0,)), ((), ())),
            preferred_element_type=jnp.float32)

    return pl.pallas_call(
        body,
        grid=(-(-_VOCAB // _PROJ_BLK),),
        in_specs=[
            pl.BlockSpec((_OUT, _HID), lambda b: (0, 0)),
            pl.BlockSpec((_HID, _PROJ_BLK), lambda b: (0, b)),
        ],
        out_specs=pl.BlockSpec((_OUT, _PROJ_BLK), lambda b: (0, b)),
        out_shape=jax.ShapeDtypeStruct((_OUT, _VPAD), jnp.float32),
    )(fc1_w, table_t)


_IVCH = _VPAD // _NW
_ICH = 2048


def _sc_interleave(ch0, ch1):
    mesh = plsc.VectorSubcoreMesh(core_axis_name="c", subcore_axis_name="s")

    @functools.partial(
        pl.kernel,
        mesh=mesh,
        compiler_params=pltpu.CompilerParams(
            use_tc_tiling_on_sc=False, needs_layout_passes=False),
        out_type=jax.ShapeDtypeStruct((_VPAD, 16), jnp.float32),
        scratch_types=[
            pltpu.VMEM((_ICH // 16, 16), jnp.float32),
            pltpu.VMEM((_ICH // 16, 16), jnp.float32),
            pltpu.VMEM((_ICH, 16), jnp.float32),
            pltpu.VMEM((_ICH, 16), jnp.float32),
            pltpu.SemaphoreType.DMA,
            pltpu.SemaphoreType.DMA,
        ],
    )
    def k(ch0_hbm, ch1_hbm, out_hbm, t0, t1, bufA, bufB, semA, semB):
        wid = lax.axis_index("s") * _NC + lax.axis_index("c")
        vbase = wid * _IVCH
        lanes = lax.iota(jnp.int32, 16)
        zcol = jnp.zeros((16,), jnp.int32)
        onecol = jnp.full((16,), 1, jnp.int32)
        zeros = jnp.zeros((16,), jnp.float32)
        bufs = (bufA, bufB)
        sems = (semA, semB)
        nch = _IVCH // _ICH

        def zero_rows(buf):
            def zero_row(r, carry):
                buf[r, pl.ds(0, 16)] = zeros
                return carry
            lax.fori_loop(0, _ICH, zero_row, 0)

        zero_rows(bufA)
        zero_rows(bufB)

        for c in range(nch):
            buf = bufs[c % 2]
            sem = sems[c % 2]
            if c >= 2:
                pltpu.make_async_copy(
                    buf, out_hbm.at[pl.ds(vbase + (c - 2) * _ICH, _ICH)],
                    sem).wait()
            q0 = (vbase + c * _ICH) // 16
            pltpu.sync_copy(ch0_hbm.at[pl.ds(q0, _ICH // 16)], t0)
            pltpu.sync_copy(ch1_hbm.at[pl.ds(q0, _ICH // 16)], t1)

            def fill(g, carry):
                rows = jnp.full((16,), g * 16, jnp.int32) + lanes
                plsc.store_scatter(buf, [rows, zcol], t0[g])
                plsc.store_scatter(buf, [rows, onecol], t1[g])
                return carry

            lax.fori_loop(0, _ICH // 16, fill, 0)
            pltpu.async_copy(
                buf, out_hbm.at[pl.ds(vbase + c * _ICH, _ICH)], sem)

        for c in (nch - 2, nch - 1):
            pltpu.make_async_copy(
                bufs[c % 2],
                out_hbm.at[pl.ds(vbase + c * _ICH, _ICH)],
                sems[c % 2]).wait()

    return k(ch0, ch1)


def _sc_pool(text_flat, proj16):
    mesh = plsc.VectorSubcoreMesh(core_axis_name="c", subcore_axis_name="s")

    @functools.partial(
        pl.kernel,
        mesh=mesh,
        compiler_params=pltpu.CompilerParams(
            use_tc_tiling_on_sc=False, needs_layout_passes=False),
        out_type=jax.ShapeDtypeStruct((_B, 16), jnp.float32),
        scratch_types=[
            pltpu.VMEM((_IPW * _L,), jnp.int32),
            pltpu.VMEM((_L, 16), jnp.float32),
            pltpu.VMEM((_L, 16), jnp.float32),
            pltpu.VMEM((_IPW, 16), jnp.float32),
            pltpu.SemaphoreType.DMA,
            pltpu.SemaphoreType.DMA,
        ],
    )
    def k(text_hbm, proj_hbm, out_hbm, idx_v, rows0, rows1, part_v,
          sem0, sem1):
        wid = lax.axis_index("s") * _NC + lax.axis_index("c")
        ibase = wid * (_IPW * _L)
        pltpu.sync_copy(text_hbm.at[pl.ds(ibase, _IPW * _L)], idx_v)

        def gather(i, buf, sem):
            off = i * _L
            pltpu.async_copy(
                proj_hbm.at[idx_v.at[pl.ds(off, _C0)]],
                buf.at[pl.ds(0, _C0)], sem)
            pltpu.async_copy(
                proj_hbm.at[idx_v.at[pl.ds(off + _C0, _C1)]],
                buf.at[pl.ds(_C0, _C1)], sem)

        def wait_gather(i, buf, sem):
            off = i * _L
            pltpu.make_async_copy(
                proj_hbm.at[idx_v.at[pl.ds(off, _C0)]],
                buf.at[pl.ds(0, _C0)], sem).wait()
            pltpu.make_async_copy(
                proj_hbm.at[idx_v.at[pl.ds(off + _C0, _C1)]],
                buf.at[pl.ds(_C0, _C1)], sem).wait()

        def reduce_to(i, buf):
            def step(s, accs):
                a = list(accs)
                for r in range(_UNROLL):
                    a[r % 4] = a[r % 4] + buf[s * _UNROLL + r]
                return tuple(a)

            accs = lax.fori_loop(
                0, _L // _UNROLL, step,
                tuple(jnp.zeros((16,), jnp.float32) for _ in range(4)))
            part_v[i, pl.ds(0, 16)] = (
                (accs[0] + accs[1]) + (accs[2] + accs[3]))

        gather(0, rows0, sem0)

        @pl.loop(0, _IPW, step=2)
        def _(i):
            gather(i + 1, rows1, sem1)
            wait_gather(i, rows0, sem0)
            reduce_to(i, rows0)

            @pl.when(i + 2 < _IPW)
            def _():
                gather(i + 2, rows0, sem0)

            wait_gather(i + 1, rows1, sem1)
            reduce_to(i + 1, rows1)

        pltpu.sync_copy(part_v, out_hbm.at[pl.ds(wid * _IPW, _IPW)])

    return k(text_flat, proj16)


def _tc_head(partial, bias):
    def body(x_ref, b_ref, o_ref):
        o_ref[...] = x_ref[:, 0:_OUT] * (1.0 / _L) + b_ref[...]

    return pl.pallas_call(
        body,
        out_shape=jax.ShapeDtypeStruct((_B, _OUT), jnp.float32),
    )(partial, bias)


def kernel(text, emb_table, fc1_w, fc1_b):
    projT = _tc_project(emb_table.T, fc1_w)
    proj16 = _sc_interleave(
        projT[0].reshape(_VPAD // 16, 16),
        projT[1].reshape(_VPAD // 16, 16),
    )
    part = _sc_pool(text.reshape(-1), proj16)
    return _tc_head(part, fc1_b.reshape(1, _OUT))


# speedup vs baseline: 1.0971x; 1.0971x over previous
import functools

import jax
import jax.numpy as jnp
from jax import lax
from jax.experimental import pallas as pl
from jax.experimental.pallas import tpu as pltpu
from jax.experimental.pallas import tpu_sc as plsc

_VOCAB = 1000000
_VPAD = 1048576
_HID = 64
_OUT = 2
_B = 4096
_L = 200

_NC = 2
_NS = 16
_NW = _NC * _NS
_IPW = _B // _NW
_C0 = 104
_C1 = _L - _C0
_UNROLL = 8

_PROJ_BLK = 65536


def _tc_project(table_t, fc1_w):
    def body(w_ref, x_ref, o_ref):
        o_ref[...] = jax.lax.dot_general(
            w_ref[...], x_ref[...], (((1,), (0,)), ((), ())),
            preferred_element_type=jnp.float32)

    return pl.pallas_call(
        body,
        grid=(-(-_VOCAB // _PROJ_BLK),),
        in_specs=[
            pl.BlockSpec((_OUT, _HID), lambda b: (0, 0)),
            pl.BlockSpec((_HID, _PROJ_BLK), lambda b: (0, b)),
        ],
        out_specs=pl.BlockSpec((_OUT, _PROJ_BLK), lambda b: (0, b)),
        out_shape=jax.ShapeDtypeStruct((_OUT, _VPAD), jnp.float32),
    )(fc1_w, table_t)


_IVCH = _VPAD // _NW
_ICH = 2048


def _sc_interleave(ch0, ch1):
    mesh = plsc.VectorSubcoreMesh(core_axis_name="c", subcore_axis_name="s")

    @functools.partial(
        pl.kernel,
        mesh=mesh,
        compiler_params=pltpu.CompilerParams(
            use_tc_tiling_on_sc=False, needs_layout_passes=False),
        out_type=jax.ShapeDtypeStruct((_VPAD, 16), jnp.float32),
        scratch_types=[
            pltpu.VMEM((_ICH // 16, 16), jnp.float32),
            pltpu.VMEM((_ICH // 16, 16), jnp.float32),
            pltpu.VMEM((_ICH // 16, 16), jnp.float32),
            pltpu.VMEM((_ICH // 16, 16), jnp.float32),
            pltpu.VMEM((_ICH, 16), jnp.float32),
            pltpu.VMEM((_ICH, 16), jnp.float32),
            pltpu.SemaphoreType.DMA,
            pltpu.SemaphoreType.DMA,
            pltpu.SemaphoreType.DMA,
            pltpu.SemaphoreType.DMA,
        ],
    )
    def k(ch0_hbm, ch1_hbm, out_hbm, t0A, t1A, t0B, t1B, bufA, bufB,
          semA, semB, stA, stB):
        wid = lax.axis_index("s") * _NC + lax.axis_index("c")
        vbase = wid * _IVCH
        lanes = lax.iota(jnp.int32, 16)
        zcol = jnp.zeros((16,), jnp.int32)
        onecol = jnp.full((16,), 1, jnp.int32)
        zeros = jnp.zeros((16,), jnp.float32)
        bufs = (bufA, bufB)
        sems = (semA, semB)
        ts = ((t0A, t1A), (t0B, t1B))
        ssems = (stA, stB)
        nch = _IVCH // _ICH
        nq = _ICH // 16

        def zero_rows(buf):
            def zero_row(r, carry):
                buf[r, pl.ds(0, 16)] = zeros
                return carry
            lax.fori_loop(0, _ICH, zero_row, 0)

        zero_rows(bufA)
        zero_rows(bufB)

        def stage(c):
            q0 = (vbase + c * _ICH) // 16
            t0, t1 = ts[c % 2]
            sem = ssems[c % 2]
            pltpu.async_copy(ch0_hbm.at[pl.ds(q0, nq)], t0, sem)
            pltpu.async_copy(ch1_hbm.at[pl.ds(q0, nq)], t1, sem)

        def wait_stage(c):
            q0 = (vbase + c * _ICH) // 16
            t0, t1 = ts[c % 2]
            sem = ssems[c % 2]
            pltpu.make_async_copy(
                ch0_hbm.at[pl.ds(q0, nq)], t0, sem).wait()
            pltpu.make_async_copy(
                ch1_hbm.at[pl.ds(q0, nq)], t1, sem).wait()

        stage(0)
        for c in range(nch):
            buf = bufs[c % 2]
            sem = sems[c % 2]
            t0, t1 = ts[c % 2]
            if c >= 2:
                pltpu.make_async_copy(
                    buf, out_hbm.at[pl.ds(vbase + (c - 2) * _ICH, _ICH)],
                    sem).wait()
            wait_stage(c)

            def fill(s, carry):
                for u in range(4):
                    g = s * 4 + u
                    rows = jnp.full((16,), g * 16, jnp.int32) + lanes
                    plsc.store_scatter(buf, [rows, zcol], t0[g])
                    plsc.store_scatter(buf, [rows, onecol], t1[g])
                return carry

            if c + 1 < nch:
                stage(c + 1)
            lax.fori_loop(0, nq // 4, fill, 0)
            pltpu.async_copy(
                buf, out_hbm.at[pl.ds(vbase + c * _ICH, _ICH)], sem)

        for c in (nch - 2, nch - 1):
            pltpu.make_async_copy(
                bufs[c % 2],
                out_hbm.at[pl.ds(vbase + c * _ICH, _ICH)],
                sems[c % 2]).wait()

    return k(ch0, ch1)


def _sc_pool(text_flat, proj16):
    mesh = plsc.VectorSubcoreMesh(core_axis_name="c", subcore_axis_name="s")

    @functools.partial(
        pl.kernel,
        mesh=mesh,
        compiler_params=pltpu.CompilerParams(
            use_tc_tiling_on_sc=False, needs_layout_passes=False),
        out_type=jax.ShapeDtypeStruct((_B, 16), jnp.float32),
        scratch_types=[
            pltpu.VMEM((_IPW * _L,), jnp.int32),
            pltpu.VMEM((_L, 16), jnp.float32),
            pltpu.VMEM((_L, 16), jnp.float32),
            pltpu.VMEM((_IPW, 16), jnp.float32),
            pltpu.SemaphoreType.DMA,
            pltpu.SemaphoreType.DMA,
        ],
    )
    def k(text_hbm, proj_hbm, out_hbm, idx_v, rows0, rows1, part_v,
          sem0, sem1):
        wid = lax.axis_index("s") * _NC + lax.axis_index("c")
        ibase = wid * (_IPW * _L)
        pltpu.sync_copy(text_hbm.at[pl.ds(ibase, _IPW * _L)], idx_v)

        def gather(i, buf, sem):
            off = i * _L
            pltpu.async_copy(
                proj_hbm.at[idx_v.at[pl.ds(off, _C0)]],
                buf.at[pl.ds(0, _C0)], sem)
            pltpu.async_copy(
                proj_hbm.at[idx_v.at[pl.ds(off + _C0, _C1)]],
                buf.at[pl.ds(_C0, _C1)], sem)

        def wait_gather(i, buf, sem):
            off = i * _L
            pltpu.make_async_copy(
                proj_hbm.at[idx_v.at[pl.ds(off, _C0)]],
                buf.at[pl.ds(0, _C0)], sem).wait()
            pltpu.make_async_copy(
                proj_hbm.at[idx_v.at[pl.ds(off + _C0, _C1)]],
                buf.at[pl.ds(_C0, _C1)], sem).wait()

        def reduce_to(i, buf):
            def step(s, accs):
                a = list(accs)
                for r in range(_UNROLL):
                    a[r % 4] = a[r % 4] + buf[s * _UNROLL + r]
                return tuple(a)

            accs = lax.fori_loop(
                0, _L // _UNROLL, step,
                tuple(jnp.zeros((16,), jnp.float32) for _ in range(4)))
            part_v[i, pl.ds(0, 16)] = (
                (accs[0] + accs[1]) + (accs[2] + accs[3]))

        gather(0, rows0, sem0)

        @pl.loop(0, _IPW, step=2)
        def _(i):
            gather(i + 1, rows1, sem1)
            wait_gather(i, rows0, sem0)
            reduce_to(i, rows0)

            @pl.when(i + 2 < _IPW)
            def _():
                gather(i + 2, rows0, sem0)

            wait_gather(i + 1, rows1, sem1)
            reduce_to(i + 1, rows1)

        pltpu.sync_copy(part_v, out_hbm.at[pl.ds(wid * _IPW, _IPW)])

    return k(text_flat, proj16)


def _tc_head(partial, bias):
    def body(x_ref, b_ref, o_ref):
        o_ref[...] = x_ref[:, 0:_OUT] * (1.0 / _L) + b_ref[...]

    return pl.pallas_call(
        body,
        out_shape=jax.ShapeDtypeStruct((_B, _OUT), jnp.float32),
    )(partial, bias)


def kernel(text, emb_table, fc1_w, fc1_b):
    projT = _tc_project(emb_table.T, fc1_w)
    proj16 = _sc_interleave(
        projT[0].reshape(_VPAD // 16, 16),
        projT[1].reshape(_VPAD // 16, 16),
    )
    part = _sc_pool(text.reshape(-1), proj16)
    return _tc_head(part, fc1_b.reshape(1, _OUT))


# speedup vs baseline: 1.2436x; 1.1335x over previous
import functools

import jax
import jax.numpy as jnp
from jax import lax
from jax.experimental import pallas as pl
from jax.experimental.pallas import tpu as pltpu
from jax.experimental.pallas import tpu_sc as plsc

_VOCAB = 1000000
_VPAD = 1048576
_HID = 64
_OUT = 2
_B = 4096
_L = 200

_NC = 2
_NS = 16
_NW = _NC * _NS
_IPW = _B // _NW
_C0 = 104
_C1 = _L - _C0
_UNROLL = 8

_PROJ_BLK = 65536


def _tc_project(table_t, fc1_w):
    def body(w_ref, x_ref, o_ref):
        o_ref[...] = jax.lax.dot_general(
            w_ref[...], x_ref[...], (((1,), (0,)), ((), ())),
            preferred_element_type=jnp.float32)

    return pl.pallas_call(
        body,
        grid=(-(-_VOCAB // _PROJ_BLK),),
        in_specs=[
            pl.BlockSpec((_OUT, _HID), lambda b: (0, 0)),
            pl.BlockSpec((_HID, _PROJ_BLK), lambda b: (0, b)),
        ],
        out_specs=pl.BlockSpec((_OUT, _PROJ_BLK), lambda b: (0, b)),
        out_shape=jax.ShapeDtypeStruct((_OUT, _VPAD), jnp.float32),
    )(fc1_w, table_t)


_IVCH = _VPAD // _NW
_ICH = 2048


def _sc_interleave(ch0, ch1):
    mesh = plsc.VectorSubcoreMesh(core_axis_name="c", subcore_axis_name="s")

    @functools.partial(
        pl.kernel,
        mesh=mesh,
        compiler_params=pltpu.CompilerParams(
            use_tc_tiling_on_sc=False, needs_layout_passes=False),
        out_type=jax.ShapeDtypeStruct((_VPAD, 16), jnp.float32),
        scratch_types=[
            pltpu.VMEM((_ICH // 16, 16), jnp.float32),
            pltpu.VMEM((_ICH // 16, 16), jnp.float32),
            pltpu.VMEM((_ICH // 16, 16), jnp.float32),
            pltpu.VMEM((_ICH // 16, 16), jnp.float32),
            pltpu.VMEM((_ICH, 16), jnp.float32),
            pltpu.VMEM((_ICH, 16), jnp.float32),
            pltpu.SemaphoreType.DMA,
            pltpu.SemaphoreType.DMA,
            pltpu.SemaphoreType.DMA,
            pltpu.SemaphoreType.DMA,
        ],
    )
    def k(ch0_hbm, ch1_hbm, out_hbm, t0A, t1A, t0B, t1B, bufA, bufB,
          semA, semB, stA, stB):
        wid = lax.axis_index("s") * _NC + lax.axis_index("c")
        vbase = wid * _IVCH
        lanes = lax.iota(jnp.int32, 16)
        zcol = jnp.zeros((16,), jnp.int32)
        onecol = jnp.full((16,), 1, jnp.int32)
        zeros = jnp.zeros((16,), jnp.float32)
        bufs = (bufA, bufB)
        sems = (semA, semB)
        ts = ((t0A, t1A), (t0B, t1B))
        ssems = (stA, stB)
        nch = _IVCH // _ICH
        nq = _ICH // 16

        def zero_rows(buf):
            def zero_row(r, carry):
                buf[r, pl.ds(0, 16)] = zeros
                return carry
            lax.fori_loop(0, _ICH, zero_row, 0)

        zero_rows(bufA)
        zero_rows(bufB)

        def stage(c):
            q0 = (vbase + c * _ICH) // 16
            t0, t1 = ts[c % 2]
            sem = ssems[c % 2]
            pltpu.async_copy(ch0_hbm.at[pl.ds(q0, nq)], t0, sem)
            pltpu.async_copy(ch1_hbm.at[pl.ds(q0, nq)], t1, sem)

        def wait_stage(c):
            q0 = (vbase + c * _ICH) // 16
            t0, t1 = ts[c % 2]
            sem = ssems[c % 2]
            pltpu.make_async_copy(
                ch0_hbm.at[pl.ds(q0, nq)], t0, sem).wait()
            pltpu.make_async_copy(
                ch1_hbm.at[pl.ds(q0, nq)], t1, sem).wait()

        stage(0)
        for c in range(nch):
            buf = bufs[c % 2]
            sem = sems[c % 2]
            t0, t1 = ts[c % 2]
            if c >= 2:
                pltpu.make_async_copy(
                    buf, out_hbm.at[pl.ds(vbase + (c - 2) * _ICH, _ICH)],
                    sem).wait()
            wait_stage(c)

            def fill(s, carry):
                for u in range(4):
                    g = s * 4 + u
                    rows = jnp.full((16,), g * 16, jnp.int32) + lanes
                    plsc.store_scatter(buf, [rows, zcol], t0[g])
                    plsc.store_scatter(buf, [rows, onecol], t1[g])
                return carry

            if c + 1 < nch:
                stage(c + 1)
            lax.fori_loop(0, nq // 4, fill, 0)
            pltpu.async_copy(
                buf, out_hbm.at[pl.ds(vbase + c * _ICH, _ICH)], sem)

        for c in (nch - 2, nch - 1):
            pltpu.make_async_copy(
                bufs[c % 2],
                out_hbm.at[pl.ds(vbase + c * _ICH, _ICH)],
                sems[c % 2]).wait()

    return k(ch0, ch1)


def _sc_pool(text_flat, proj16):
    mesh = plsc.VectorSubcoreMesh(core_axis_name="c", subcore_axis_name="s")

    @functools.partial(
        pl.kernel,
        mesh=mesh,
        compiler_params=pltpu.CompilerParams(
            use_tc_tiling_on_sc=False, needs_layout_passes=False),
        out_type=jax.ShapeDtypeStruct((_B, 16), jnp.float32),
        scratch_types=[
            pltpu.VMEM((_IPW * _L,), jnp.int32),
            pltpu.VMEM((_L, 16), jnp.float32),
            pltpu.VMEM((_L, 16), jnp.float32),
            pltpu.VMEM((_L, 16), jnp.float32),
            pltpu.VMEM((_L, 16), jnp.float32),
            pltpu.VMEM((_IPW, 16), jnp.float32),
            pltpu.SemaphoreType.DMA,
            pltpu.SemaphoreType.DMA,
            pltpu.SemaphoreType.DMA,
            pltpu.SemaphoreType.DMA,
        ],
    )
    def k(text_hbm, proj_hbm, out_hbm, idx_v, rows0, rows1, rows2, rows3,
          part_v, sem0, sem1, sem2, sem3):
        wid = lax.axis_index("s") * _NC + lax.axis_index("c")
        ibase = wid * (_IPW * _L)
        pltpu.sync_copy(text_hbm.at[pl.ds(ibase, _IPW * _L)], idx_v)

        def gather(i, buf, sem):
            off = i * _L
            pltpu.async_copy(
                proj_hbm.at[idx_v.at[pl.ds(off, _C0)]],
                buf.at[pl.ds(0, _C0)], sem)
            pltpu.async_copy(
                proj_hbm.at[idx_v.at[pl.ds(off + _C0, _C1)]],
                buf.at[pl.ds(_C0, _C1)], sem)

        def wait_gather(i, buf, sem):
            off = i * _L
            pltpu.make_async_copy(
                proj_hbm.at[idx_v.at[pl.ds(off, _C0)]],
                buf.at[pl.ds(0, _C0)], sem).wait()
            pltpu.make_async_copy(
                proj_hbm.at[idx_v.at[pl.ds(off + _C0, _C1)]],
                buf.at[pl.ds(_C0, _C1)], sem).wait()

        def reduce_to(i, buf):
            def step(s, accs):
                a = list(accs)
                for r in range(_UNROLL):
                    a[r % 4] = a[r % 4] + buf[s * _UNROLL + r]
                return tuple(a)

            accs = lax.fori_loop(
                0, _L // _UNROLL, step,
                tuple(jnp.zeros((16,), jnp.float32) for _ in range(4)))
            part_v[i, pl.ds(0, 16)] = (
                (accs[0] + accs[1]) + (accs[2] + accs[3]))

        ring = ((rows0, sem0), (rows1, sem1), (rows2, sem2), (rows3, sem3))
        for j in range(3):
            gather(j, ring[j][0], ring[j][1])

        @pl.loop(0, _IPW, step=4)
        def _(i):
            for j in range(4):
                buf, sem = ring[j]

                @pl.when(i + j + 3 < _IPW)
                def _():
                    nbuf, nsem = ring[(j + 3) % 4]
                    gather(i + j + 3, nbuf, nsem)

                wait_gather(i + j, buf, sem)
                reduce_to(i + j, buf)

        pltpu.sync_copy(part_v, out_hbm.at[pl.ds(wid * _IPW, _IPW)])

    return k(text_flat, proj16)


def _tc_head(partial, bias):
    def body(x_ref, b_ref, o_ref):
        o_ref[...] = x_ref[:, 0:_OUT] * (1.0 / _L) + b_ref[...]

    return pl.pallas_call(
        body,
        out_shape=jax.ShapeDtypeStruct((_B, _OUT), jnp.float32),
    )(partial, bias)


def kernel(text, emb_table, fc1_w, fc1_b):
    projT = _tc_project(emb_table.T, fc1_w)
    proj16 = _sc_interleave(
        projT[0].reshape(_VPAD // 16, 16),
        projT[1].reshape(_VPAD // 16, 16),
    )
    part = _sc_pool(text.reshape(-1), proj16)
    return _tc_head(part, fc1_b.reshape(1, _OUT))
